# baseline (speedup 1.0000x reference)
import jax
import jax.numpy as jnp
from jax import lax
from jax.experimental import pallas as pl
from jax.experimental.pallas import tpu as pltpu
from jax.experimental.pallas import tpu_sc as plsc

_B = 4096
_L = 20
_D = 64
_LANES = 16
_NC = 2
_NS = 16
_NW = _NC * _NS
_BPW = _B // _NW
_G = _BPW // _LANES
_L2 = 1e-05

_N = 100000
_NSLAB = 782
_SPW = 25
_NL = _NSLAB * 64


def _repack_slab(tab_hbm, out_hbm, s, par, stg_v, otile_v, ssem, osem, iota):
    pltpu.make_async_copy(
        tab_hbm.at[:, pl.ds(s * 128, 128)],
        stg_v.at[pl.ds(par * 64, 64)], ssem.at[par]).wait()

    pbase = par * 64
    iota8 = iota * 8

    def fbody(f4, carry):
        for u0 in range(8):
            uvec = iota8 + u0
            lr_v = lax.shift_right_logical(uvec, 1) + pbase
            colc = (u0 & 1) * _D
            for df in range(8):
                f0 = f4 * 8 + df
                fvec = lax.bitwise_and(iota8 + f0, _D - 1)
                v = plsc.load_gather(stg_v, [fvec + pbase, uvec])
                plsc.store_scatter(otile_v, [lr_v, fvec + colc], v)
        return carry
    lax.fori_loop(0, _D // 8, fbody, jnp.int32(0))

    pltpu.async_copy(otile_v.at[pl.ds(par * 64, 64)],
                     out_hbm.at[pl.ds(s * 64, 64)], osem.at[par])


def _stage_slab(tab_hbm, s, par, stg_v, ssem):
    pltpu.async_copy(
        tab_hbm.at[:, pl.ds(s * 128, 128)],
        stg_v.at[pl.ds(par * 64, 64)], ssem.at[par])


def _repack_body(uft_hbm, bft_hbm, tailu_hbm, tailb_hbm, ufp_hbm, bfp_hbm,
                 stg_v, otile_v, ssem, osem):
    cid = lax.axis_index("c")
    sid = lax.axis_index("s")
    wid = sid * _NC + cid
    iota = lax.iota(jnp.int32, _LANES)

    for tab_hbm, out_hbm in ((uft_hbm, ufp_hbm), (bft_hbm, bfp_hbm)):
        _stage_slab(tab_hbm, wid, 0, stg_v, ssem)

        def body(k, carry):
            s = wid + 32 * k
            par = lax.rem(k, 2)
            nxt = 1 - par

            @pl.when(s < _NSLAB - 1)
            def _go():
                @pl.when(s + 32 < _NSLAB - 1)
                def _prefetch():
                    _stage_slab(tab_hbm, s + 32, nxt, stg_v, ssem)

                @pl.when(k >= 2)
                def _drain():
                    pltpu.make_async_copy(
                        otile_v.at[pl.ds(par * 64, 64)],
                        out_hbm.at[pl.ds((s - 64) * 64, 64)],
                        osem.at[par]).wait()
                _repack_slab(tab_hbm, out_hbm, s, par,
                             stg_v, otile_v, ssem, osem, iota)
            return carry
        lax.fori_loop(0, _SPW, body, jnp.int32(0))

        def drain(k, carry):
            s = wid + 32 * k
            par = lax.rem(k, 2)

            @pl.when(jnp.logical_and(s < _NSLAB - 1,
                                     jnp.logical_or(s + 64 >= _NSLAB - 1,
                                                    k + 2 >= _SPW)))
            def _d():
                pltpu.make_async_copy(
                    otile_v.at[pl.ds(par * 64, 64)],
                    out_hbm.at[pl.ds(s * 64, 64)], osem.at[par]).wait()
            return carry
        lax.fori_loop(0, _SPW, drain, jnp.int32(0))

    @pl.when(wid == 31)
    def _tail_u():
        pltpu.sync_copy(tailu_hbm, otile_v.at[pl.ds(0, 16)])
        pltpu.sync_copy(otile_v.at[pl.ds(0, 16)],
                        ufp_hbm.at[pl.ds((_NSLAB - 1) * 64, 16)])

    @pl.when(wid == 30)
    def _tail_b():
        pltpu.sync_copy(tailb_hbm, otile_v.at[pl.ds(64, 16)])
        pltpu.sync_copy(otile_v.at[pl.ds(64, 16)],
                        bfp_hbm.at[pl.ds((_NSLAB - 1) * 64, 16)])


_repack_kernel = pl.kernel(
    _repack_body,
    out_type=[
        jax.ShapeDtypeStruct((_NL, 2 * _D), jnp.float32),
        jax.ShapeDtypeStruct((_NL, 2 * _D), jnp.float32),
    ],
    mesh=plsc.VectorSubcoreMesh(core_axis_name="c", subcore_axis_name="s"),
    compiler_params=pltpu.CompilerParams(
        needs_layout_passes=False, use_tc_tiling_on_sc=True),
    scratch_types=[
        pltpu.VMEM((128, 128), jnp.float32),
        pltpu.VMEM((128, 128), jnp.float32),
        pltpu.SemaphoreType.DMA((2,)),
        pltpu.SemaphoreType.DMA((2,)),
    ],
)


def _line_of(u):
    line = lax.shift_right_logical(u, 1)
    hoff = lax.bitwise_and(u, 1) * _D
    return line, hoff


def _sc_body(users_hbm, bundles_hbm, ufp_hbm, bfp_hbm,
             pred_hbm, part_hbm,
             uidx_v, uoff_v, bidx_v, idxt_v, offt_v,
             urows_v, brows_v, pbuf_v, pvec_v, usem, bsem):
    cid = lax.axis_index("c")
    sid = lax.axis_index("s")
    wid = sid * _NC + cid
    base = wid * _BPW
    iota = lax.iota(jnp.int32, _LANES)

    pltpu.sync_copy(users_hbm.at[pl.ds(base, _BPW)], uidx_v)
    pltpu.sync_copy(bundles_hbm.at[pl.ds(base * _L, _BPW * _L)], bidx_v)

    for g in range(_G):
        u = uidx_v[pl.ds(g * _LANES, _LANES)]
        line, hoff = _line_of(u)
        uidx_v[pl.ds(g * _LANES, _LANES)] = line
        uoff_v[pl.ds(g * _LANES, _LANES)] = hoff

    udma = pltpu.async_copy(ufp_hbm.at[uidx_v], urows_v, usem)

    def tbody(l, carry):
        ls = jnp.full((_LANES,), l, jnp.int32)
        for g in range(_G):
            src = (iota + (g * _LANES)) * _L + ls
            u = plsc.load_gather(bidx_v, [src])
            line, hoff = _line_of(u)
            dst = pl.ds(l * _BPW + g * _LANES, _LANES)
            idxt_v[dst] = line
            offt_v[dst] = hoff
        return carry
    lax.fori_loop(0, _L, tbody, jnp.int32(0))

    pltpu.async_copy(
        bfp_hbm.at[idxt_v.at[pl.ds(0, _BPW)]],
        brows_v.at[pl.ds(0, _BPW)], bsem.at[0])

    udma.wait()

    iota8 = iota * 8

    def ubody(g, usq):
        rows = iota + (g * _LANES)
        hv = uoff_v[pl.ds(g * _LANES, _LANES)]
        for j in range(_D):
            cols = lax.bitwise_and(iota8 + j, _D - 1)
            v = plsc.load_gather(urows_v, [rows, hv + cols])
            usq = usq + v * v
        return usq
    usq = lax.fori_loop(0, _G, ubody, jnp.zeros((_LANES,), jnp.float32))

    def body(l, bsq):
        cur = lax.rem(l, 2)
        nxt = 1 - cur

        @pl.when(l + 1 < _L)
        def _prefetch():
            pltpu.async_copy(
                bfp_hbm.at[idxt_v.at[pl.ds((l + 1) * _BPW, _BPW)]],
                brows_v.at[pl.ds(nxt * _BPW, _BPW)], bsem.at[nxt])

        pltpu.make_async_copy(
            bfp_hbm.at[idxt_v.at[pl.ds(l * _BPW, _BPW)]],
            brows_v.at[pl.ds(cur * _BPW, _BPW)], bsem.at[cur]).wait()

        ls = jnp.full((_LANES,), l, jnp.int32)
        roff = cur * _BPW
        rows_g = [iota + (g * _LANES) for g in range(_G)]
        uhv_g = [uoff_v[pl.ds(g * _LANES, _LANES)] for g in range(_G)]
        bhv_g = [offt_v[pl.ds(l * _BPW + g * _LANES, _LANES)]
                 for g in range(_G)]
        acc_g = [jnp.zeros((_LANES,), jnp.float32) for _ in range(_G)]
        bsq_g = [jnp.zeros((_LANES,), jnp.float32) for _ in range(_G)]
        def jbody(j4, carry):
            acc_c, bsq_c = carry
            for dj in range(4):
                cols = lax.bitwise_and(iota8 + (j4 * 4 + dj), _D - 1)
                for g in range(_G):
                    uv = plsc.load_gather(
                        urows_v, [rows_g[g], uhv_g[g] + cols])
                    bv = plsc.load_gather(
                        brows_v, [rows_g[g] + roff, bhv_g[g] + cols])
                    acc_c[g] = acc_c[g] + uv * bv
                    bsq_c[g] = bsq_c[g] + bv * bv
            return acc_c, bsq_c
        acc_g, bsq_g = lax.fori_loop(
            0, _D // 4, jbody, (acc_g, bsq_g))
        for g in range(_G):
            plsc.store_scatter(pbuf_v, [rows_g[g] * _L + ls], acc_g[g])
        for g in range(_G):
            bsq = bsq + bsq_g[g]
        return bsq
    bsq = lax.fori_loop(0, _L, body, jnp.zeros((_LANES,), jnp.float32))

    pltpu.sync_copy(pbuf_v, pred_hbm.at[pl.ds(base * _L, _BPW * _L)])

    pvec_v[...] = jnp.float32(_L) * usq + bsq
    pltpu.sync_copy(pvec_v, part_hbm.at[pl.ds(wid * _LANES, _LANES)])


_sc_kernel = pl.kernel(
    _sc_body,
    out_type=[
        jax.ShapeDtypeStruct((_B * _L,), jnp.float32),
        jax.ShapeDtypeStruct((_NW * _LANES,), jnp.float32),
    ],
    mesh=plsc.VectorSubcoreMesh(core_axis_name="c", subcore_axis_name="s"),
    compiler_params=pltpu.CompilerParams(
        needs_layout_passes=False, use_tc_tiling_on_sc=False),
    scratch_types=[
        pltpu.VMEM((_BPW,), jnp.int32),
        pltpu.VMEM((_BPW,), jnp.int32),
        pltpu.VMEM((_BPW * _L,), jnp.int32),
        pltpu.VMEM((_L * _BPW,), jnp.int32),
        pltpu.VMEM((_L * _BPW,), jnp.int32),
        pltpu.VMEM((_BPW, 2 * _D), jnp.float32),
        pltpu.VMEM((2 * _BPW, 2 * _D), jnp.float32),
        pltpu.VMEM((_BPW * _L,), jnp.float32),
        pltpu.VMEM((_LANES,), jnp.float32),
        pltpu.SemaphoreType.DMA,
        pltpu.SemaphoreType.DMA((2,)),
    ],
)


@jax.jit
def kernel(users, bundles, users_feature, bundles_feature):
    tailu = users_feature[(_NSLAB - 1) * 128:].reshape(16, 128)
    tailb = bundles_feature[(_NSLAB - 1) * 128:].reshape(16, 128)
    ufp, bfp = _repack_kernel(
        users_feature.T, bundles_feature.T, tailu, tailb)
    pred_flat, parts = _sc_kernel(
        users.reshape(_B), bundles.reshape(_B * _L), ufp, bfp)
    pred = pred_flat.reshape(_B, _L)
    loss = jnp.float32(_L2) * jnp.sum(parts)
    return (pred, loss)


# speedup vs baseline: 1.4340x; 1.4340x over previous
import jax
import jax.numpy as jnp
from jax import lax
from jax.experimental import pallas as pl
from jax.experimental.pallas import tpu as pltpu
from jax.experimental.pallas import tpu_sc as plsc

_B = 4096
_L = 20
_D = 64
_LANES = 16
_NC = 2
_NS = 16
_NW = _NC * _NS
_BPW = _B // _NW
_G = _BPW // _LANES
_L2 = 1e-05


def _sc_body(users_hbm, bundles_hbm, uf_hbm, bf_hbm,
             pred_hbm, part_hbm,
             uidx_v, bidx_v, idxt_v, urows_v, brows_v, pbuf_v, pvec_v,
             usem, bsem):
    cid = lax.axis_index("c")
    sid = lax.axis_index("s")
    wid = sid * _NC + cid
    base = wid * _BPW
    iota = lax.iota(jnp.int32, _LANES)
    skew = lax.shift_right_logical(iota, 1) * 8

    pltpu.sync_copy(users_hbm.at[pl.ds(base, _BPW)], uidx_v)
    pltpu.sync_copy(bundles_hbm.at[pl.ds(base * _L, _BPW * _L)], bidx_v)

    udma = pltpu.async_copy(uf_hbm.at[uidx_v], urows_v, usem)

    def tbody(l, carry):
        ls = jnp.full((_LANES,), l, jnp.int32)
        for g in range(_G):
            src = (iota + (g * _LANES)) * _L + ls
            idxt_v[pl.ds(l * _BPW + g * _LANES, _LANES)] = (
                plsc.load_gather(bidx_v, [src]))
        return carry
    lax.fori_loop(0, _L, tbody, jnp.int32(0))

    pltpu.async_copy(
        bf_hbm.at[idxt_v.at[pl.ds(0, _BPW)]],
        brows_v.at[pl.ds(0, _BPW)], bsem.at[0])

    udma.wait()

    def ubody(g, usq):
        rows = iota + (g * _LANES)
        u0 = jnp.zeros((_LANES,), jnp.float32)
        u1 = jnp.zeros((_LANES,), jnp.float32)
        for j in range(0, _D, 2):
            c0 = lax.bitwise_and(skew + j, _D - 1)
            c1 = lax.bitwise_and(skew + (j + 1), _D - 1)
            v0 = plsc.load_gather(urows_v, [rows, c0])
            v1 = plsc.load_gather(urows_v, [rows, c1])
            u0 = u0 + v0 * v0
            u1 = u1 + v1 * v1
        return usq + u0 + u1
    usq = lax.fori_loop(0, _G, ubody, jnp.zeros((_LANES,), jnp.float32))

    def body(l, bsq):
        cur = lax.rem(l, 2)
        nxt = 1 - cur

        @pl.when(l + 1 < _L)
        def _prefetch():
            pltpu.async_copy(
                bf_hbm.at[idxt_v.at[pl.ds((l + 1) * _BPW, _BPW)]],
                brows_v.at[pl.ds(nxt * _BPW, _BPW)], bsem.at[nxt])

        pltpu.make_async_copy(
            bf_hbm.at[idxt_v.at[pl.ds(l * _BPW, _BPW)]],
            brows_v.at[pl.ds(cur * _BPW, _BPW)], bsem.at[cur]).wait()

        ls = jnp.full((_LANES,), l, jnp.int32)
        roff = cur * _BPW

        def gbody(g, bsq):
            rows = iota + (g * _LANES)
            brows = rows + roff
            a0 = jnp.zeros((_LANES,), jnp.float32)
            a1 = jnp.zeros((_LANES,), jnp.float32)
            b0 = jnp.zeros((_LANES,), jnp.float32)
            b1 = jnp.zeros((_LANES,), jnp.float32)
            for j in range(0, _D, 2):
                c0 = lax.bitwise_and(skew + j, _D - 1)
                c1 = lax.bitwise_and(skew + (j + 1), _D - 1)
                uv0 = plsc.load_gather(urows_v, [rows, c0])
                bv0 = plsc.load_gather(brows_v, [brows, c0])
                uv1 = plsc.load_gather(urows_v, [rows, c1])
                bv1 = plsc.load_gather(brows_v, [brows, c1])
                a0 = a0 + uv0 * bv0
                b0 = b0 + bv0 * bv0
                a1 = a1 + uv1 * bv1
                b1 = b1 + bv1 * bv1
            plsc.store_scatter(pbuf_v, [rows * _L + ls], a0 + a1)
            return bsq + (b0 + b1)
        return lax.fori_loop(0, _G, gbody, bsq)
    bsq = lax.fori_loop(0, _L, body, jnp.zeros((_LANES,), jnp.float32))

    pltpu.sync_copy(pbuf_v, pred_hbm.at[pl.ds(base * _L, _BPW * _L)])

    pvec_v[...] = jnp.float32(_L) * usq + bsq
    pltpu.sync_copy(pvec_v, part_hbm.at[pl.ds(wid * _LANES, _LANES)])


_sc_kernel = pl.kernel(
    _sc_body,
    out_type=[
        jax.ShapeDtypeStruct((_B * _L,), jnp.float32),
        jax.ShapeDtypeStruct((_NW * _LANES,), jnp.float32),
    ],
    mesh=plsc.VectorSubcoreMesh(core_axis_name="c", subcore_axis_name="s"),
    compiler_params=pltpu.CompilerParams(
        needs_layout_passes=False, use_tc_tiling_on_sc=False),
    scratch_types=[
        pltpu.VMEM((_BPW,), jnp.int32),
        pltpu.VMEM((_BPW * _L,), jnp.int32),
        pltpu.VMEM((_L * _BPW,), jnp.int32),
        pltpu.VMEM((_BPW, _D), jnp.float32),
        pltpu.VMEM((2 * _BPW, _D), jnp.float32),
        pltpu.VMEM((_BPW * _L,), jnp.float32),
        pltpu.VMEM((_LANES,), jnp.float32),
        pltpu.SemaphoreType.DMA,
        pltpu.SemaphoreType.DMA((2,)),
    ],
)


@jax.jit
def kernel(users, bundles, users_feature, bundles_feature):
    pred_flat, parts = _sc_kernel(
        users.reshape(_B), bundles.reshape(_B * _L),
        users_feature, bundles_feature)
    pred = pred_flat.reshape(_B, _L)
    loss = jnp.float32(_L2) * jnp.sum(parts)
    return (pred, loss)
